# speedup vs baseline: 2.8137x; 2.8137x over previous
import functools

import jax
import jax.numpy as jnp
from jax import lax
from jax.experimental import pallas as pl
from jax.experimental.pallas import tpu as pltpu

N = 50000
E = 800000
F = 32
H = 512
G = 128

NP = 50176
EP = 802816
PAD_NODE = NP - 1


def _prop(h, src, dst, u):
    y = h * u
    t = y + jnp.zeros_like(y).at[dst].add(y[src])
    return t * u


def kernel(x, edge_index, batch, W1, b1, W2, b2, W3, b3, Wl, bl):
    f32 = jnp.float32
    src = jnp.full((EP,), PAD_NODE, jnp.int32).at[:E].set(edge_index[0])
    dst = jnp.full((EP,), PAD_NODE, jnp.int32).at[:E].set(edge_index[1])
    xp = jnp.zeros((NP, F), f32).at[:N].set(x)
    batch_p = jnp.full((NP,), G, jnp.int32).at[:N].set(batch)

    deg = jnp.ones((NP,), f32).at[dst].add(1.0)
    u = (deg ** -0.5)[:, None]

    z1 = _prop(xp, src, dst, u)
    h1 = jax.nn.relu(z1 @ W1 + b1)
    z2 = _prop(h1, src, dst, u)
    h2 = jax.nn.relu(z2 @ W2 + b2)
    s = h2 @ (W3 @ Wl)
    z3 = _prop(s, src, dst, u)

    oh = (batch_p[:, None] == jnp.arange(G)[None, :]).astype(f32)
    sums = oh.T @ z3
    cnt = jnp.sum(oh, axis=0)[:, None]
    out = sums / jnp.clip(cnt, 1.0, None) + (cnt > 0) * (b3 @ Wl)[0] + bl
    return out


# speedup vs baseline: 4.8073x; 1.7085x over previous
import functools

import jax
import jax.numpy as jnp
from jax import lax
from jax.experimental import pallas as pl
from jax.experimental.pallas import tpu as pltpu
from jax.experimental.pallas import tpu_sc as plsc

N = 50000
E = 800000
F = 32
H = 512
G = 128

NP = 50176
EP = 819200
PAD_NODE = NP - 1
ER = EP // 128
ER_W = ER // 32
CH = 40
NW = 32

_mesh = functools.partial(
    pl.kernel,
    mesh=plsc.VectorSubcoreMesh(core_axis_name="c", subcore_axis_name="s"),
    compiler_params=pltpu.CompilerParams(needs_layout_passes=False),
)

f32 = jnp.float32
i32 = jnp.int32


def _sc_prop1(vals, src2d, dst2d, zeros_n, count_only):
    ins = (src2d, dst2d, zeros_n) if count_only else (vals, src2d, dst2d, zeros_n)

    def body(*refs):
        if count_only:
            src_h, dst_h, zeros_h, out_h, acc, sbuf, dbuf = refs
            yloc = None
        else:
            vals_h, src_h, dst_h, zeros_h, out_h, acc, yloc, sbuf, dbuf = refs
        wid = lax.axis_index("s") * 2 + lax.axis_index("c")
        pltpu.sync_copy(zeros_h, acc)
        if yloc is not None:
            pltpu.sync_copy(vals_h, yloc)
        row0 = wid * ER_W
        ones = jnp.full((16,), 1.0, f32)

        def chunk(ci, _):
            pltpu.sync_copy(src_h.at[pl.ds(row0 + ci * CH, CH)], sbuf)
            pltpu.sync_copy(dst_h.at[pl.ds(row0 + ci * CH, CH)], dbuf)

            def row(ri, _):
                for k in range(8):
                    d = dbuf[ri, pl.ds(k * 16, 16)]
                    if yloc is None:
                        v = ones
                    else:
                        s = sbuf[ri, pl.ds(k * 16, 16)]
                        v = plsc.load_gather(yloc, [s])
                    plsc.addupdate_scatter(acc, [d], v)
                return 0

            lax.fori_loop(0, CH, row, 0)
            return 0

        lax.fori_loop(0, ER_W // CH, chunk, 0)
        pltpu.sync_copy(acc, out_h.at[pl.ds(wid * NP, NP)])

    scratch = [pltpu.VMEM((NP,), f32)]
    if not count_only:
        scratch.append(pltpu.VMEM((NP,), f32))
    scratch += [pltpu.VMEM((CH, 128), i32), pltpu.VMEM((CH, 128), i32)]
    fn = _mesh(
        body,
        out_type=jax.ShapeDtypeStruct((NW * NP,), f32),
        scratch_types=scratch,
        name="sc_prop1_cnt" if count_only else "sc_prop1",
    )
    return fn(*ins)


def _uy0_body(degp_ref, x_ref, u_ref, y0_ref):
    deg = 1.0 + jnp.sum(degp_ref[...], axis=0)[:, None]
    u = lax.rsqrt(deg)
    u_ref[...] = u
    y0_ref[...] = x_ref[...] * u


def _tc_u_y0(degp, xp):
    return pl.pallas_call(
        _uy0_body,
        grid=(NP // 256,),
        in_specs=[
            pl.BlockSpec((NW, 256), lambda i: (0, i)),
            pl.BlockSpec((256, F), lambda i: (i, 0)),
        ],
        out_specs=[
            pl.BlockSpec((256, 1), lambda i: (i, 0)),
            pl.BlockSpec((256, F), lambda i: (i, 0)),
        ],
        out_shape=[
            jax.ShapeDtypeStruct((NP, 1), f32),
            jax.ShapeDtypeStruct((NP, F), f32),
        ],
    )(degp, xp)


def _pool_body(z3p_ref, y2_ref, u_ref, b_ref, b3_ref, Wl_ref, bl_ref, out_ref,
               sums, cnt):
    i = pl.program_id(0)

    @pl.when(i == 0)
    def _():
        sums[...] = jnp.zeros_like(sums)
        cnt[...] = jnp.zeros_like(cnt)

    t3 = y2_ref[...] + jnp.sum(z3p_ref[...], axis=0)[:, None]
    z3 = t3 * u_ref[...]
    gids = lax.broadcasted_iota(i32, (256, G), 1)
    oh = (b_ref[...] == gids).astype(f32)
    sums[...] += lax.dot_general(oh, z3, (((0,), (0,)), ((), ())),
                                 preferred_element_type=f32)
    cnt[...] += jnp.sum(oh, axis=0)[:, None]

    @pl.when(i == NP // 256 - 1)
    def _():
        c = jnp.dot(b3_ref[...], Wl_ref[...], preferred_element_type=f32)[0, 0]
        cv = cnt[...]
        out_ref[...] = (sums[...] / jnp.clip(cv, 1.0, None)
                        + (cv > 0).astype(f32) * c + bl_ref[0, 0])


def _tc_pool(z3p, y2, u, batch_col, b3, Wl, bl):
    return pl.pallas_call(
        _pool_body,
        grid=(NP // 256,),
        in_specs=[
            pl.BlockSpec((NW, 256), lambda i: (0, i)),
            pl.BlockSpec((256, 1), lambda i: (i, 0)),
            pl.BlockSpec((256, 1), lambda i: (i, 0)),
            pl.BlockSpec((256, 1), lambda i: (i, 0)),
            pl.BlockSpec((1, H), lambda i: (0, 0)),
            pl.BlockSpec((H, 1), lambda i: (0, 0)),
            pl.BlockSpec((1, 1), lambda i: (0, 0)),
        ],
        out_specs=pl.BlockSpec((G, 1), lambda i: (0, 0)),
        out_shape=jax.ShapeDtypeStruct((G, 1), f32),
        scratch_shapes=[pltpu.VMEM((G, 1), f32), pltpu.VMEM((G, 1), f32)],
    )(z3p, y2, u, batch_col, b3, Wl, bl)


def kernel(x, edge_index, batch, W1, b1, W2, b2, W3, b3, Wl, bl):
    src = jnp.full((EP,), PAD_NODE, i32).at[:E].set(edge_index[0])
    dst = jnp.full((EP,), PAD_NODE, i32).at[:E].set(edge_index[1])
    src2d = src.reshape(ER, 128)
    dst2d = dst.reshape(ER, 128)
    xp = jnp.zeros((NP, F), f32).at[:N].set(x)
    batch_col = jnp.full((NP, 1), G, i32).at[:N, 0].set(batch)
    zeros_n = jnp.zeros((NP,), f32)

    degp = _sc_prop1(None, src2d, dst2d, zeros_n,
                     count_only=True).reshape(NW, NP)
    u, y0 = _tc_u_y0(degp, xp)

    t1 = y0 + jnp.zeros_like(y0).at[dst].add(y0[src])
    h1 = jax.nn.relu((t1 * u) @ W1 + b1)
    y1 = h1 * u
    t2 = y1 + jnp.zeros_like(y1).at[dst].add(y1[src])
    h2 = jax.nn.relu((t2 * u) @ W2 + b2)
    y2 = (h2 @ (W3 @ Wl)) * u

    z3p = _sc_prop1(y2[:, 0], src2d, dst2d, zeros_n,
                     count_only=False).reshape(NW, NP)
    return _tc_pool(z3p, y2, u, batch_col, b3.reshape(1, H), Wl,
                    bl.reshape(1, 1))


# speedup vs baseline: 9.5125x; 1.9788x over previous
import functools

import jax
import jax.numpy as jnp
from jax import lax
from jax.experimental import pallas as pl
from jax.experimental.pallas import tpu as pltpu
from jax.experimental.pallas import tpu_sc as plsc

N = 50000
E = 800000
F = 32
H = 512
G = 128

NP = 50176
EP = 819200
PAD_NODE = NP - 1
ER = EP // 128
ER_W = ER // 32
CH = 40
NW = 32

_mesh = functools.partial(
    pl.kernel,
    mesh=plsc.VectorSubcoreMesh(core_axis_name="c", subcore_axis_name="s"),
    compiler_params=pltpu.CompilerParams(needs_layout_passes=False, use_tc_tiling_on_sc=False),
)

f32 = jnp.float32
i32 = jnp.int32


def _sc_prop1(vals, src2d, dst2d, zeros_n, count_only):
    ins = (src2d, dst2d, zeros_n) if count_only else (vals, src2d, dst2d, zeros_n)

    def body(*refs):
        if count_only:
            src_h, dst_h, zeros_h, out_h, acc, sbuf, dbuf = refs
            yloc = None
        else:
            vals_h, src_h, dst_h, zeros_h, out_h, acc, yloc, sbuf, dbuf = refs
        wid = lax.axis_index("s") * 2 + lax.axis_index("c")
        pltpu.sync_copy(zeros_h, acc)
        if yloc is not None:
            pltpu.sync_copy(vals_h, yloc)
        row0 = wid * ER_W
        ones = jnp.full((16,), 1.0, f32)

        def chunk(ci, _):
            pltpu.sync_copy(src_h.at[pl.ds(row0 + ci * CH, CH)], sbuf)
            pltpu.sync_copy(dst_h.at[pl.ds(row0 + ci * CH, CH)], dbuf)

            def row(ri, _):
                for k in range(8):
                    d = dbuf[ri, pl.ds(k * 16, 16)]
                    if yloc is None:
                        v = ones
                    else:
                        s = sbuf[ri, pl.ds(k * 16, 16)]
                        v = plsc.load_gather(yloc, [s])
                    plsc.addupdate_scatter(acc, [d], v)
                return 0

            lax.fori_loop(0, CH, row, 0)
            return 0

        lax.fori_loop(0, ER_W // CH, chunk, 0)
        pltpu.sync_copy(acc, out_h.at[pl.ds(wid * NP, NP)])

    scratch = [pltpu.VMEM((NP,), f32)]
    if not count_only:
        scratch.append(pltpu.VMEM((NP,), f32))
    scratch += [pltpu.VMEM((CH, 128), i32), pltpu.VMEM((CH, 128), i32)]
    fn = _mesh(
        body,
        out_type=jax.ShapeDtypeStruct((NW * NP,), f32),
        scratch_types=scratch,
        name="sc_prop1_cnt" if count_only else "sc_prop1",
    )
    return fn(*ins)


def _sc_prop32(y0, src2d, dst2d):
    def body(y_h, src_h, dst_h, outA, outB, spmem, rows_v, sbuf, dbuf, sem):
        c = lax.axis_index("c")
        s = lax.axis_index("s")
        r0 = s * (NP // 16)
        pltpu.sync_copy(y_h.at[pl.ds(r0, NP // 16)],
                        spmem.at[pl.ds(r0, NP // 16)])
        plsc.subcore_barrier()
        erow0 = (c * 16 + s) * ER_W

        def chunk(ci, _):
            pltpu.sync_copy(src_h.at[pl.ds(erow0 + ci * 8, 8)], sbuf)
            pltpu.sync_copy(dst_h.at[pl.ds(erow0 + ci * 8, 8)], dbuf)

            def row(ri, _):
                pltpu.async_copy(y_h.at[sbuf.at[ri]], rows_v, sem).wait()
                pltpu.sync_copy(rows_v, spmem.at[dbuf.at[ri]], add=True)
                return 0

            lax.fori_loop(0, 8, row, 0)
            return 0

        lax.fori_loop(0, ER_W // 8, chunk, 0)
        plsc.subcore_barrier()

        @pl.when(c == 0)
        def _():
            pltpu.sync_copy(spmem.at[pl.ds(r0, NP // 16)],
                            outA.at[pl.ds(r0, NP // 16)])

        @pl.when(c == 1)
        def _():
            pltpu.sync_copy(spmem.at[pl.ds(r0, NP // 16)],
                            outB.at[pl.ds(r0, NP // 16)])

    fn = _mesh(
        body,
        out_type=[jax.ShapeDtypeStruct((NP, F), f32),
                  jax.ShapeDtypeStruct((NP, F), f32)],
        scratch_types=[
            pltpu.VMEM_SHARED((NP, F), f32),
            pltpu.VMEM((128, F), f32),
            pltpu.VMEM((8, 128), i32),
            pltpu.VMEM((8, 128), i32),
            pltpu.SemaphoreType.DMA,
        ],
        name="sc_prop32",
    )
    return fn(y0, src2d, dst2d)


BINR = 2816
NBINS = 18
FL = 64
EW16 = ER // 16

def _sc_prop512(y1, src2d, dst2d):
    def body(y_h, src_h, dst_h, out_h, spmem, rows_v, ebs, ebd,
             sbuf, obuf, sidx, oidx, sem):
        c = lax.axis_index("c")
        s = lax.axis_index("s")
        erow0 = s * EW16
        lane = jnp.arange(16, dtype=i32)

        def flush(scnt):
            for j in range(FL // 16):
                sidx[0, pl.ds(j * 16, 16)] = sbuf[0, pl.ds(j * 16, 16)]
                oidx[0, pl.ds(j * 16, 16)] = obuf[0, pl.ds(j * 16, 16)]
            for j in range(8):
                sbuf[0, pl.ds(j * 16, 16)] = sbuf[0, pl.ds(FL + j * 16, 16)]
                obuf[0, pl.ds(j * 16, 16)] = obuf[0, pl.ds(FL + j * 16, 16)]
            pltpu.async_copy(y_h.at[sidx.at[0]], rows_v, sem).wait()
            pltpu.sync_copy(rows_v, spmem.at[oidx.at[0]], add=True)
            return scnt - FL

        for b in range(NBINS):
            lo = b * BINR
            nrows = min(BINR, NP - lo)

            full, rem = nrows // 256, nrows % 256

            def _bin_copy(dst_of_src):
                @pl.when(s < full)
                def _():
                    dst_of_src(s * 256, 256)
                if rem:
                    @pl.when(s == full)
                    def _():
                        dst_of_src(full * 256, rem)

            @pl.when(b % 2 == c)
            def _():
                _bin_copy(lambda o, n: pltpu.sync_copy(
                    y_h.at[pl.ds(lo + o, n)], spmem.at[pl.ds(o, n)]))
                plsc.subcore_barrier()

                def chunk(ci, scnt):
                    pltpu.sync_copy(src_h.at[pl.ds(erow0 + ci * 8, 8)], ebs)
                    pltpu.sync_copy(dst_h.at[pl.ds(erow0 + ci * 8, 8)], ebd)

                    def row(ri, scnt):
                        for k in range(8):
                            d = ebd[ri, pl.ds(k * 16, 16)]
                            sv = ebs[ri, pl.ds(k * 16, 16)]
                            m = (d >= lo) & (d < lo + nrows)
                            plsc.store_compressed(
                                sbuf.at[0, pl.ds(scnt, 16)], sv, mask=m)
                            plsc.store_compressed(
                                obuf.at[0, pl.ds(scnt, 16)], d - lo, mask=m)
                            scnt = scnt + jnp.sum(m.astype(i32))
                        return lax.while_loop(
                            lambda t: t >= FL, flush, scnt)

                    return lax.fori_loop(0, 8, row, scnt)

                scnt = lax.fori_loop(0, EW16 // 8, chunk, jnp.int32(0))

                @pl.when(scnt > 0)
                def _():
                    for j in range(FL // 16):
                        valid = (lane + j * 16) < scnt
                        o = jnp.where(valid, obuf[0, pl.ds(j * 16, 16)],
                                      jnp.int32(BINR))
                        sv = jnp.where(valid, sbuf[0, pl.ds(j * 16, 16)],
                                       jnp.int32(0))
                        obuf[0, pl.ds(j * 16, 16)] = o
                        sbuf[0, pl.ds(j * 16, 16)] = sv
                    flush(scnt)

                plsc.subcore_barrier()
                _bin_copy(lambda o, n: pltpu.sync_copy(
                    spmem.at[pl.ds(o, n)], out_h.at[pl.ds(lo + o, n)]))
                plsc.subcore_barrier()

    fn = _mesh(
        body,
        out_type=jax.ShapeDtypeStruct((NP, H), f32),
        scratch_types=[
            pltpu.VMEM_SHARED((BINR + 1, H), f32),
            pltpu.VMEM((FL, H), f32),
            pltpu.VMEM((8, 128), i32),
            pltpu.VMEM((8, 128), i32),
            pltpu.VMEM((1, 208), i32),
            pltpu.VMEM((1, 208), i32),
            pltpu.VMEM((1, FL), i32),
            pltpu.VMEM((1, FL), i32),
            pltpu.SemaphoreType.DMA,
        ],
        name="sc_prop512",
    )
    return fn(y1, src2d, dst2d)


def _uy0_body(degp_ref, x_ref, u_ref, y0_ref):
    deg = 1.0 + jnp.sum(degp_ref[...], axis=0)[:, None]
    u = lax.rsqrt(deg)
    u_ref[...] = u
    y0_ref[...] = x_ref[...] * u


def _tc_u_y0(degp, xp):
    return pl.pallas_call(
        _uy0_body,
        grid=(NP // 256,),
        in_specs=[
            pl.BlockSpec((NW, 256), lambda i: (0, i)),
            pl.BlockSpec((256, F), lambda i: (i, 0)),
        ],
        out_specs=[
            pl.BlockSpec((256, 1), lambda i: (i, 0)),
            pl.BlockSpec((256, F), lambda i: (i, 0)),
        ],
        out_shape=[
            jax.ShapeDtypeStruct((NP, 1), f32),
            jax.ShapeDtypeStruct((NP, F), f32),
        ],
    )(degp, xp)


def _l1_body(tA_ref, tB_ref, y0_ref, u_ref, W1_ref, b1_ref, y1_ref):
    u = u_ref[...]
    z1 = (tA_ref[...] + tB_ref[...] - y0_ref[...]) * u
    h1 = jnp.maximum(jnp.dot(z1, W1_ref[...], preferred_element_type=f32)
                     + b1_ref[...], 0.0)
    y1_ref[...] = h1 * u


def _tc_layer1(tA, tB, y0, u, W1, b1):
    return pl.pallas_call(
        _l1_body,
        grid=(NP // 512,),
        in_specs=[
            pl.BlockSpec((512, F), lambda i: (i, 0)),
            pl.BlockSpec((512, F), lambda i: (i, 0)),
            pl.BlockSpec((512, F), lambda i: (i, 0)),
            pl.BlockSpec((512, 1), lambda i: (i, 0)),
            pl.BlockSpec((F, H), lambda i: (0, 0)),
            pl.BlockSpec((1, H), lambda i: (0, 0)),
        ],
        out_specs=pl.BlockSpec((512, H), lambda i: (i, 0)),
        out_shape=jax.ShapeDtypeStruct((NP, H), f32),
    )(tA, tB, y0, u, W1, b1)


def _l2_body(t2_ref, u_ref, W2_ref, b2_ref, W3_ref, Wl_ref, y2_ref):
    u = u_ref[...]
    z2 = t2_ref[...] * u
    h2 = jnp.maximum(jnp.dot(z2, W2_ref[...], preferred_element_type=f32)
                     + b2_ref[...], 0.0)
    v = jnp.dot(W3_ref[...], Wl_ref[...], preferred_element_type=f32)
    y2_ref[...] = jnp.dot(h2, v, preferred_element_type=f32) * u


def _tc_layer2(t2, u, W2, b2, W3, Wl):
    return pl.pallas_call(
        _l2_body,
        grid=(NP // 512,),
        in_specs=[
            pl.BlockSpec((512, H), lambda i: (i, 0)),
            pl.BlockSpec((512, 1), lambda i: (i, 0)),
            pl.BlockSpec((H, H), lambda i: (0, 0)),
            pl.BlockSpec((1, H), lambda i: (0, 0)),
            pl.BlockSpec((H, H), lambda i: (0, 0)),
            pl.BlockSpec((H, 1), lambda i: (0, 0)),
        ],
        out_specs=pl.BlockSpec((512, 1), lambda i: (i, 0)),
        out_shape=jax.ShapeDtypeStruct((NP, 1), f32),
    )(t2, u, W2, b2, W3, Wl)


def _pool_body(z3p_ref, y2_ref, u_ref, b_ref, b3_ref, Wl_ref, bl_ref, out_ref,
               sums, cnt):
    i = pl.program_id(0)

    @pl.when(i == 0)
    def _():
        sums[...] = jnp.zeros_like(sums)
        cnt[...] = jnp.zeros_like(cnt)

    t3 = y2_ref[...] + jnp.sum(z3p_ref[...], axis=0)[:, None]
    z3 = t3 * u_ref[...]
    gids = lax.broadcasted_iota(i32, (256, G), 1)
    oh = (b_ref[...] == gids).astype(f32)
    sums[...] += lax.dot_general(oh, z3, (((0,), (0,)), ((), ())),
                                 preferred_element_type=f32)
    cnt[...] += jnp.sum(oh, axis=0)[:, None]

    @pl.when(i == NP // 256 - 1)
    def _():
        c = jnp.dot(b3_ref[...], Wl_ref[...], preferred_element_type=f32)[0, 0]
        cv = cnt[...]
        out_ref[...] = (sums[...] / jnp.clip(cv, 1.0, None)
                        + (cv > 0).astype(f32) * c + bl_ref[0, 0])


def _tc_pool(z3p, y2, u, batch_col, b3, Wl, bl):
    return pl.pallas_call(
        _pool_body,
        grid=(NP // 256,),
        in_specs=[
            pl.BlockSpec((NW, 256), lambda i: (0, i)),
            pl.BlockSpec((256, 1), lambda i: (i, 0)),
            pl.BlockSpec((256, 1), lambda i: (i, 0)),
            pl.BlockSpec((256, 1), lambda i: (i, 0)),
            pl.BlockSpec((1, H), lambda i: (0, 0)),
            pl.BlockSpec((H, 1), lambda i: (0, 0)),
            pl.BlockSpec((1, 1), lambda i: (0, 0)),
        ],
        out_specs=pl.BlockSpec((G, 1), lambda i: (0, 0)),
        out_shape=jax.ShapeDtypeStruct((G, 1), f32),
        scratch_shapes=[pltpu.VMEM((G, 1), f32), pltpu.VMEM((G, 1), f32)],
    )(z3p, y2, u, batch_col, b3, Wl, bl)


def kernel(x, edge_index, batch, W1, b1, W2, b2, W3, b3, Wl, bl):
    src = jnp.full((EP,), PAD_NODE, i32).at[:E].set(edge_index[0])
    dst = jnp.full((EP,), PAD_NODE, i32).at[:E].set(edge_index[1])
    src2d = src.reshape(ER, 128)
    dst2d = dst.reshape(ER, 128)
    xp = jnp.zeros((NP, F), f32).at[:N].set(x)
    batch_col = jnp.full((NP, 1), G, i32).at[:N, 0].set(batch)
    zeros_n = jnp.zeros((NP,), f32)

    degp = _sc_prop1(None, src2d, dst2d, zeros_n,
                     count_only=True).reshape(NW, NP)
    u, y0 = _tc_u_y0(degp, xp)

    tA, tB = _sc_prop32(y0, src2d, dst2d)
    y1 = _tc_layer1(tA, tB, y0, u, W1, b1.reshape(1, H))

    t2 = _sc_prop512(y1, src2d, dst2d)
    y2 = _tc_layer2(t2, u, W2, b2.reshape(1, H), W3, Wl)

    z3p = _sc_prop1(y2[:, 0], src2d, dst2d, zeros_n,
                     count_only=False).reshape(NW, NP)
    return _tc_pool(z3p, y2, u, batch_col, b3.reshape(1, H), Wl,
                    bl.reshape(1, 1))


# speedup vs baseline: 11.7096x; 1.2310x over previous
import functools

import jax
import jax.numpy as jnp
from jax import lax
from jax.experimental import pallas as pl
from jax.experimental.pallas import tpu as pltpu
from jax.experimental.pallas import tpu_sc as plsc

N = 50000
E = 800000
F = 32
H = 512
G = 128

NP = 50176
EP = 819200
PAD_NODE = NP - 1
ER = EP // 128
ER_W = ER // 32
CH = 40
NW = 32

_mesh = functools.partial(
    pl.kernel,
    mesh=plsc.VectorSubcoreMesh(core_axis_name="c", subcore_axis_name="s"),
    compiler_params=pltpu.CompilerParams(needs_layout_passes=False, use_tc_tiling_on_sc=False),
)

f32 = jnp.float32
i32 = jnp.int32


def _sc_prop1(vals, src2d, dst2d, zeros_n, count_only):
    ins = (src2d, dst2d, zeros_n) if count_only else (vals, src2d, dst2d, zeros_n)

    def body(*refs):
        if count_only:
            src_h, dst_h, zeros_h, out_h, acc, sbuf, dbuf = refs
            yloc = None
        else:
            vals_h, src_h, dst_h, zeros_h, out_h, acc, yloc, sbuf, dbuf = refs
        wid = lax.axis_index("s") * 2 + lax.axis_index("c")
        pltpu.sync_copy(zeros_h, acc)
        if yloc is not None:
            pltpu.sync_copy(vals_h, yloc)
        row0 = wid * ER_W
        ones = jnp.full((16,), 1.0, f32)

        def chunk(ci, _):
            pltpu.sync_copy(src_h.at[pl.ds(row0 + ci * CH, CH)], sbuf)
            pltpu.sync_copy(dst_h.at[pl.ds(row0 + ci * CH, CH)], dbuf)

            def row(ri, _):
                for k in range(8):
                    d = dbuf[ri, pl.ds(k * 16, 16)]
                    if yloc is None:
                        v = ones
                    else:
                        s = sbuf[ri, pl.ds(k * 16, 16)]
                        v = plsc.load_gather(yloc, [s])
                    plsc.addupdate_scatter(acc, [d], v)
                return 0

            lax.fori_loop(0, CH, row, 0)
            return 0

        lax.fori_loop(0, ER_W // CH, chunk, 0)
        pltpu.sync_copy(acc, out_h.at[pl.ds(wid * NP, NP)])

    scratch = [pltpu.VMEM((NP,), f32)]
    if not count_only:
        scratch.append(pltpu.VMEM((NP,), f32))
    scratch += [pltpu.VMEM((CH, 128), i32), pltpu.VMEM((CH, 128), i32)]
    fn = _mesh(
        body,
        out_type=jax.ShapeDtypeStruct((NW * NP,), f32),
        scratch_types=scratch,
        name="sc_prop1_cnt" if count_only else "sc_prop1",
    )
    return fn(*ins)


def _sc_prop32(y0, src2d, dst2d):
    def body(y_h, src_h, dst_h, outA, outB, spmem, rows_v, sbuf, dbuf, sem):
        c = lax.axis_index("c")
        s = lax.axis_index("s")
        r0 = s * (NP // 16)
        pltpu.sync_copy(y_h.at[pl.ds(r0, NP // 16)],
                        spmem.at[pl.ds(r0, NP // 16)])
        plsc.subcore_barrier()
        erow0 = (c * 16 + s) * ER_W

        def chunk(ci, _):
            pltpu.sync_copy(src_h.at[pl.ds(erow0 + ci * 8, 8)], sbuf)
            pltpu.sync_copy(dst_h.at[pl.ds(erow0 + ci * 8, 8)], dbuf)

            def row(ri, _):
                pltpu.async_copy(y_h.at[sbuf.at[ri]], rows_v, sem).wait()
                pltpu.sync_copy(rows_v, spmem.at[dbuf.at[ri]], add=True)
                return 0

            lax.fori_loop(0, 8, row, 0)
            return 0

        lax.fori_loop(0, ER_W // 8, chunk, 0)
        plsc.subcore_barrier()

        @pl.when(c == 0)
        def _():
            pltpu.sync_copy(spmem.at[pl.ds(r0, NP // 16)],
                            outA.at[pl.ds(r0, NP // 16)])

        @pl.when(c == 1)
        def _():
            pltpu.sync_copy(spmem.at[pl.ds(r0, NP // 16)],
                            outB.at[pl.ds(r0, NP // 16)])

    fn = _mesh(
        body,
        out_type=[jax.ShapeDtypeStruct((NP, F), f32),
                  jax.ShapeDtypeStruct((NP, F), f32)],
        scratch_types=[
            pltpu.VMEM_SHARED((NP, F), f32),
            pltpu.VMEM((128, F), f32),
            pltpu.VMEM((8, 128), i32),
            pltpu.VMEM((8, 128), i32),
            pltpu.SemaphoreType.DMA,
        ],
        name="sc_prop32",
    )
    return fn(y0, src2d, dst2d)


BINR = 2560
NBINS = 20
FL = 32
EW16 = ER // 16

def _sc_prop512(y1, src2d, dst2d):
    def body(y_h, src_h, dst_h, out_h, spmem, rows0, rows1, ebs, ebd,
             sbuf, obuf, sidx0, oidx0, sidx1, oidx1, sem0, sem1):
        c = lax.axis_index("c")
        s = lax.axis_index("s")
        erow0 = s * EW16
        lane = jnp.arange(16, dtype=i32)
        slots = ((sidx0, oidx0, rows0, sem0), (sidx1, oidx1, rows1, sem1))

        def wait_add(p):
            si, oi, rv, sm = slots[p]
            pltpu.make_async_copy(y_h.at[si.at[0]], rv, sm).wait()
            pltpu.sync_copy(rv, spmem.at[oi.at[0]], add=True)

        def flush(scnt, fcount):
            p = fcount % 2

            for q in (0, 1):
                @pl.when((fcount >= 2) & (p == q))
                def _():
                    wait_add(q)

            for q in (0, 1):
                @pl.when(p == q)
                def _():
                    si, oi, rv, sm = slots[q]
                    for j in range(FL // 16):
                        si[0, pl.ds(j * 16, 16)] = sbuf[0, pl.ds(j * 16, 16)]
                        oi[0, pl.ds(j * 16, 16)] = obuf[0, pl.ds(j * 16, 16)]

            for j in range(8):
                sbuf[0, pl.ds(j * 16, 16)] = sbuf[0, pl.ds(FL + j * 16, 16)]
                obuf[0, pl.ds(j * 16, 16)] = obuf[0, pl.ds(FL + j * 16, 16)]

            for q in (0, 1):
                @pl.when(p == q)
                def _():
                    si, oi, rv, sm = slots[q]
                    pltpu.async_copy(y_h.at[si.at[0]], rv, sm)
            return scnt - FL, fcount + 1

        def flush_w(st):
            return flush(st[0], st[1])

        for b in range(NBINS):
            lo = b * BINR
            nrows = min(BINR, NP - lo)
            full = nrows // 256
            assert nrows % 256 == 0

            def _bin_copy(fn):
                @pl.when(s < full)
                def _():
                    fn(s * 256, 256)

            @pl.when(b % 2 == c)
            def _():
                _bin_copy(lambda o, n: pltpu.sync_copy(
                    y_h.at[pl.ds(lo + o, n)], spmem.at[pl.ds(o, n)]))
                plsc.subcore_barrier()

                def chunk(ci, st):
                    pltpu.sync_copy(src_h.at[pl.ds(erow0 + ci * 8, 8)], ebs)
                    pltpu.sync_copy(dst_h.at[pl.ds(erow0 + ci * 8, 8)], ebd)

                    def row(ri, st):
                        scnt, fcount = st
                        for k in range(8):
                            d = ebd[ri, pl.ds(k * 16, 16)]
                            sv = ebs[ri, pl.ds(k * 16, 16)]
                            m = (d >= lo) & (d < lo + nrows)
                            plsc.store_compressed(
                                sbuf.at[0, pl.ds(scnt, 16)], sv, mask=m)
                            plsc.store_compressed(
                                obuf.at[0, pl.ds(scnt, 16)], d - lo, mask=m)
                            scnt = scnt + jnp.sum(m.astype(i32))
                        return lax.while_loop(
                            lambda t: t[0] >= FL, flush_w, (scnt, fcount))

                    return lax.fori_loop(0, 8, row, st)

                scnt, fcount = lax.fori_loop(
                    0, EW16 // 8, chunk, (jnp.int32(0), jnp.int32(0)))

                @pl.when(scnt > 0)
                def _():
                    for j in range(FL // 16):
                        valid = (lane + j * 16) < scnt
                        o = jnp.where(valid, obuf[0, pl.ds(j * 16, 16)],
                                      jnp.int32(BINR))
                        sv = jnp.where(valid, sbuf[0, pl.ds(j * 16, 16)],
                                       jnp.int32(0))
                        obuf[0, pl.ds(j * 16, 16)] = o
                        sbuf[0, pl.ds(j * 16, 16)] = sv

                fcount = lax.cond(scnt > 0,
                                  lambda: flush(scnt, fcount)[1],
                                  lambda: fcount)
                for off in (2, 1):
                    @pl.when(fcount >= off)
                    def _():
                        pq = (fcount - off) % 2
                        for q in (0, 1):
                            @pl.when(pq == q)
                            def _():
                                wait_add(q)

                plsc.subcore_barrier()
                _bin_copy(lambda o, n: pltpu.sync_copy(
                    spmem.at[pl.ds(o, n)], out_h.at[pl.ds(lo + o, n)]))
                plsc.subcore_barrier()

    fn = _mesh(
        body,
        out_type=jax.ShapeDtypeStruct((NP, H), f32),
        scratch_types=[
            pltpu.VMEM_SHARED((BINR + 1, H), f32),
            pltpu.VMEM((FL, H), f32),
            pltpu.VMEM((FL, H), f32),
            pltpu.VMEM((8, 128), i32),
            pltpu.VMEM((8, 128), i32),
            pltpu.VMEM((1, 176), i32),
            pltpu.VMEM((1, 176), i32),
            pltpu.VMEM((1, FL), i32),
            pltpu.VMEM((1, FL), i32),
            pltpu.VMEM((1, FL), i32),
            pltpu.VMEM((1, FL), i32),
            pltpu.SemaphoreType.DMA,
            pltpu.SemaphoreType.DMA,
        ],
        name="sc_prop512",
    )
    return fn(y1, src2d, dst2d)


def _uy0_body(degp_ref, x_ref, u_ref, y0_ref):
    deg = 1.0 + jnp.sum(degp_ref[...], axis=0)[:, None]
    u = lax.rsqrt(deg)
    u_ref[...] = u
    y0_ref[...] = x_ref[...] * u


def _tc_u_y0(degp, xp):
    return pl.pallas_call(
        _uy0_body,
        grid=(NP // 256,),
        in_specs=[
            pl.BlockSpec((NW, 256), lambda i: (0, i)),
            pl.BlockSpec((256, F), lambda i: (i, 0)),
        ],
        out_specs=[
            pl.BlockSpec((256, 1), lambda i: (i, 0)),
            pl.BlockSpec((256, F), lambda i: (i, 0)),
        ],
        out_shape=[
            jax.ShapeDtypeStruct((NP, 1), f32),
            jax.ShapeDtypeStruct((NP, F), f32),
        ],
    )(degp, xp)


def _l1_body(tA_ref, tB_ref, y0_ref, u_ref, W1_ref, b1_ref, y1_ref):
    u = u_ref[...]
    z1 = (tA_ref[...] + tB_ref[...] - y0_ref[...]) * u
    h1 = jnp.maximum(jnp.dot(z1, W1_ref[...], preferred_element_type=f32)
                     + b1_ref[...], 0.0)
    y1_ref[...] = h1 * u


def _tc_layer1(tA, tB, y0, u, W1, b1):
    return pl.pallas_call(
        _l1_body,
        grid=(NP // 512,),
        in_specs=[
            pl.BlockSpec((512, F), lambda i: (i, 0)),
            pl.BlockSpec((512, F), lambda i: (i, 0)),
            pl.BlockSpec((512, F), lambda i: (i, 0)),
            pl.BlockSpec((512, 1), lambda i: (i, 0)),
            pl.BlockSpec((F, H), lambda i: (0, 0)),
            pl.BlockSpec((1, H), lambda i: (0, 0)),
        ],
        out_specs=pl.BlockSpec((512, H), lambda i: (i, 0)),
        out_shape=jax.ShapeDtypeStruct((NP, H), f32),
    )(tA, tB, y0, u, W1, b1)


def _l2_body(t2_ref, u_ref, W2_ref, b2_ref, W3_ref, Wl_ref, y2_ref):
    u = u_ref[...]
    z2 = t2_ref[...] * u
    h2 = jnp.maximum(jnp.dot(z2, W2_ref[...], preferred_element_type=f32)
                     + b2_ref[...], 0.0)
    v = jnp.dot(W3_ref[...], Wl_ref[...], preferred_element_type=f32)
    y2_ref[...] = jnp.dot(h2, v, preferred_element_type=f32) * u


def _tc_layer2(t2, u, W2, b2, W3, Wl):
    return pl.pallas_call(
        _l2_body,
        grid=(NP // 512,),
        in_specs=[
            pl.BlockSpec((512, H), lambda i: (i, 0)),
            pl.BlockSpec((512, 1), lambda i: (i, 0)),
            pl.BlockSpec((H, H), lambda i: (0, 0)),
            pl.BlockSpec((1, H), lambda i: (0, 0)),
            pl.BlockSpec((H, H), lambda i: (0, 0)),
            pl.BlockSpec((H, 1), lambda i: (0, 0)),
        ],
        out_specs=pl.BlockSpec((512, 1), lambda i: (i, 0)),
        out_shape=jax.ShapeDtypeStruct((NP, 1), f32),
    )(t2, u, W2, b2, W3, Wl)


def _pool_body(z3p_ref, y2_ref, u_ref, b_ref, b3_ref, Wl_ref, bl_ref, out_ref,
               sums, cnt):
    i = pl.program_id(0)

    @pl.when(i == 0)
    def _():
        sums[...] = jnp.zeros_like(sums)
        cnt[...] = jnp.zeros_like(cnt)

    t3 = y2_ref[...] + jnp.sum(z3p_ref[...], axis=0)[:, None]
    z3 = t3 * u_ref[...]
    gids = lax.broadcasted_iota(i32, (256, G), 1)
    oh = (b_ref[...] == gids).astype(f32)
    sums[...] += lax.dot_general(oh, z3, (((0,), (0,)), ((), ())),
                                 preferred_element_type=f32)
    cnt[...] += jnp.sum(oh, axis=0)[:, None]

    @pl.when(i == NP // 256 - 1)
    def _():
        c = jnp.dot(b3_ref[...], Wl_ref[...], preferred_element_type=f32)[0, 0]
        cv = cnt[...]
        out_ref[...] = (sums[...] / jnp.clip(cv, 1.0, None)
                        + (cv > 0).astype(f32) * c + bl_ref[0, 0])


def _tc_pool(z3p, y2, u, batch_col, b3, Wl, bl):
    return pl.pallas_call(
        _pool_body,
        grid=(NP // 256,),
        in_specs=[
            pl.BlockSpec((NW, 256), lambda i: (0, i)),
            pl.BlockSpec((256, 1), lambda i: (i, 0)),
            pl.BlockSpec((256, 1), lambda i: (i, 0)),
            pl.BlockSpec((256, 1), lambda i: (i, 0)),
            pl.BlockSpec((1, H), lambda i: (0, 0)),
            pl.BlockSpec((H, 1), lambda i: (0, 0)),
            pl.BlockSpec((1, 1), lambda i: (0, 0)),
        ],
        out_specs=pl.BlockSpec((G, 1), lambda i: (0, 0)),
        out_shape=jax.ShapeDtypeStruct((G, 1), f32),
        scratch_shapes=[pltpu.VMEM((G, 1), f32), pltpu.VMEM((G, 1), f32)],
    )(z3p, y2, u, batch_col, b3, Wl, bl)


def kernel(x, edge_index, batch, W1, b1, W2, b2, W3, b3, Wl, bl):
    src = jnp.full((EP,), PAD_NODE, i32).at[:E].set(edge_index[0])
    dst = jnp.full((EP,), PAD_NODE, i32).at[:E].set(edge_index[1])
    src2d = src.reshape(ER, 128)
    dst2d = dst.reshape(ER, 128)
    xp = jnp.zeros((NP, F), f32).at[:N].set(x)
    batch_col = jnp.full((NP, 1), G, i32).at[:N, 0].set(batch)
    zeros_n = jnp.zeros((NP,), f32)

    degp = _sc_prop1(None, src2d, dst2d, zeros_n,
                     count_only=True).reshape(NW, NP)
    u, y0 = _tc_u_y0(degp, xp)

    tA, tB = _sc_prop32(y0, src2d, dst2d)
    y1 = _tc_layer1(tA, tB, y0, u, W1, b1.reshape(1, H))

    t2 = _sc_prop512(y1, src2d, dst2d)
    y2 = _tc_layer2(t2, u, W2, b2.reshape(1, H), W3, Wl)

    z3p = _sc_prop1(y2[:, 0], src2d, dst2d, zeros_n,
                     count_only=False).reshape(NW, NP)
    return _tc_pool(z3p, y2, u, batch_col, b3.reshape(1, H), Wl,
                    bl.reshape(1, 1))
